# baseline (speedup 1.0000x reference)
import functools

import jax
import jax.numpy as jnp
from jax import lax
from jax.experimental import pallas as pl
from jax.experimental.pallas import tpu as pltpu
from jax.experimental.pallas import tpu_sc as plsc

NUM_BINS = 255
MIN_V = -20.0
MAX_V = 20.0
BIN_WIDTH = (MAX_V - MIN_V) / (NUM_BINS - 1)

N = 262144
N_SC = 172032
N_TC = N - N_SC

NC = 2
NS = 16
NW = NC * NS
RW = N_SC // NW
C = 64
NCH = RW // C
NBUF = 4
L = 16

BR = 1024


def _sc_body(values_hbm, out_hbm, vals_v, *rest):
    bufs = rest[:NBUF]
    idxs = rest[NBUF:2 * NBUF]
    sems = rest[2 * NBUF:3 * NBUF]

    wid = lax.axis_index("s") * NC + lax.axis_index("c")
    row0 = wid * RW

    pltpu.sync_copy(values_hbm.at[pl.ds(row0, RW)], vals_v)

    zeros = jnp.zeros((L,), jnp.float32)
    lane = lax.iota(jnp.int32, L)

    def memset_rows(buf):
        def body(r, carry):
            for g in range(NUM_BINS // L):
                buf[r, pl.ds(g * L, L)] = zeros
            buf[r, pl.ds(NUM_BINS - L, L)] = zeros
            return carry
        lax.fori_loop(0, C, body, 0)

    for b in range(NBUF):
        memset_rows(bufs[b])

    def process(chunk, b):
        buf = bufs[b]
        idx = idxs[b]
        vbase = chunk * C
        for g in range(C // L):
            v = vals_v[pl.ds(vbase + g * L, L)]
            v = jnp.minimum(jnp.maximum(v, MIN_V), MAX_V)
            norm = (v - MIN_V) / BIN_WIDTH
            lo = norm.astype(jnp.int32)
            lo = jnp.minimum(lo, NUM_BINS - 1)
            lof = lo.astype(jnp.float32)
            up = jnp.where(norm > lof, lo + 1, lo)
            up = jnp.minimum(up, NUM_BINS - 1)
            uw = norm - lof
            lw = 1.0 - uw
            rows = lane + (g * L)
            plsc.store_scatter(buf, [rows, lo], lw)
            plsc.addupdate_scatter(buf, [rows, up], uw)
            idx[pl.ds(g * L, L)] = lo
            idx[pl.ds(C + g * L, L)] = up

    def issue(chunk, b):
        dst = out_hbm.at[pl.ds(row0 + chunk * C, C)]
        pltpu.async_copy(bufs[b], dst, sems[b])

    def drain(chunk, b):
        dst = out_hbm.at[pl.ds(row0 + chunk * C, C)]
        pltpu.make_async_copy(bufs[b], dst, sems[b]).wait()

    for b in range(NBUF):
        process(b, b)
        issue(b, b)

    def ring_body(p, carry):
        for b in range(NBUF):
            chunk = p * NBUF + b
            drain(chunk - NBUF, b)
            for g in range(C // L):
                rows = lane + (g * L)
                plsc.store_scatter(bufs[b], [rows, idxs[b][pl.ds(g * L, L)]], zeros)
                plsc.store_scatter(bufs[b], [rows, idxs[b][pl.ds(C + g * L, L)]], zeros)
            process(chunk, b)
            issue(chunk, b)
        return carry

    lax.fori_loop(1, NCH // NBUF, ring_body, 0)

    for b in range(NBUF):
        drain(NCH - NBUF + b, b)


@functools.partial(
    pl.kernel,
    out_type=jax.ShapeDtypeStruct((N_SC, NUM_BINS), jnp.float32),
    mesh=plsc.VectorSubcoreMesh(core_axis_name="c", subcore_axis_name="s"),
    compiler_params=pltpu.CompilerParams(needs_layout_passes=False),
    scratch_types=(
        [pltpu.VMEM((RW,), jnp.float32)]
        + [pltpu.VMEM((C, NUM_BINS), jnp.float32) for _ in range(NBUF)]
        + [pltpu.VMEM((2 * C,), jnp.int32) for _ in range(NBUF)]
        + [pltpu.SemaphoreType.DMA for _ in range(NBUF)]
    ),
)
def _two_hot_sc(values_hbm, out_hbm, vals_v, *rest):
    _sc_body(values_hbm, out_hbm, vals_v, *rest)


def _tc_body(vals_ref, out_ref):
    v = vals_ref[:]
    v = jnp.minimum(jnp.maximum(v, MIN_V), MAX_V)
    norm = (v - MIN_V) / BIN_WIDTH
    norm2 = lax.broadcast_in_dim(norm, (BR, NUM_BINS), (0,))
    colf = lax.broadcasted_iota(jnp.int32, (BR, NUM_BINS), 1).astype(jnp.float32)
    out_ref[...] = jnp.maximum(1.0 - jnp.abs(norm2 - colf), 0.0)


_tc_call = pl.pallas_call(
    _tc_body,
    out_shape=jax.ShapeDtypeStruct((N_TC, NUM_BINS), jnp.float32),
    grid=(N_TC // BR,),
    in_specs=[pl.BlockSpec((BR,), lambda i: (i + N_SC // BR,))],
    out_specs=pl.BlockSpec((BR, NUM_BINS), lambda i: (i, 0)),
)


def kernel(values):
    head = _two_hot_sc(values)
    tail = _tc_call(values)
    return jnp.concatenate([head, tail], axis=0)


# speedup vs baseline: 2.5753x; 2.5753x over previous
import functools

import jax
import jax.numpy as jnp
from jax import lax
from jax.experimental import pallas as pl
from jax.experimental.pallas import tpu as pltpu
from jax.experimental.pallas import tpu_sc as plsc

NUM_BINS = 255
MIN_V = -20.0
MAX_V = 20.0
BIN_WIDTH = (MAX_V - MIN_V) / (NUM_BINS - 1)

N = 262144
NC = 2
NS = 16
NW = NC * NS
RW = N // NW
C = 128
NCH = RW // C
NBUF = 2
L = 16


def _sc_body(values_hbm, out_hbm, vals_v, *rest):
    bufs = rest[:NBUF]
    idxs = rest[NBUF:2 * NBUF]
    sems = rest[2 * NBUF:3 * NBUF]
    vsem = rest[3 * NBUF]

    wid = lax.axis_index("s") * NC + lax.axis_index("c")
    row0 = wid * RW

    vcopy = pltpu.async_copy(values_hbm.at[pl.ds(row0, RW)], vals_v, vsem)

    zeros = jnp.zeros((L,), jnp.float32)
    lane = lax.iota(jnp.int32, L)

    def memset_body(r, carry):
        for buf in bufs:
            for g in range(NUM_BINS // L):
                buf[r, pl.ds(g * L, L)] = zeros
            buf[r, pl.ds(NUM_BINS - L, L)] = zeros
        return carry

    lax.fori_loop(0, C, memset_body, 0)
    vcopy.wait()

    def process(chunk, b):
        buf = bufs[b]
        idx = idxs[b]
        vbase = chunk * C

        def group(g, carry):
            v = vals_v[pl.ds(vbase + g * L, L)]
            v = jnp.minimum(jnp.maximum(v, MIN_V), MAX_V)
            norm = (v - MIN_V) / BIN_WIDTH
            lo = norm.astype(jnp.int32)
            lo = jnp.minimum(lo, NUM_BINS - 1)
            lof = lo.astype(jnp.float32)
            up = jnp.where(norm > lof, lo + 1, lo)
            up = jnp.minimum(up, NUM_BINS - 1)
            uw = norm - lof
            lw = 1.0 - uw
            rows = lane + g * L
            plsc.store_scatter(buf, [rows, lo], lw)
            plsc.addupdate_scatter(buf, [rows, up], uw)
            idx[pl.ds(g * L, L)] = lo
            idx[pl.ds(C + g * L, L)] = up
            return carry

        lax.fori_loop(0, C // L, group, 0)

    def clear_prev(b):
        def group(g, carry):
            rows = lane + g * L
            plsc.store_scatter(bufs[b], [rows, idxs[b][pl.ds(g * L, L)]], zeros)
            plsc.store_scatter(bufs[b], [rows, idxs[b][pl.ds(C + g * L, L)]], zeros)
            return carry

        lax.fori_loop(0, C // L, group, 0)

    def issue(chunk, b):
        dst = out_hbm.at[pl.ds(row0 + chunk * C, C)]
        pltpu.async_copy(bufs[b], dst, sems[b])

    def drain(chunk, b):
        dst = out_hbm.at[pl.ds(row0 + chunk * C, C)]
        pltpu.make_async_copy(bufs[b], dst, sems[b]).wait()

    for b in range(NBUF):
        process(b, b)
        issue(b, b)

    def ring_body(p, carry):
        for b in range(NBUF):
            chunk = p * NBUF + b
            drain(chunk - NBUF, b)
            clear_prev(b)
            process(chunk, b)
            issue(chunk, b)
        return carry

    lax.fori_loop(1, NCH // NBUF, ring_body, 0)

    for b in range(NBUF):
        drain(NCH - NBUF + b, b)


@functools.partial(
    pl.kernel,
    out_type=jax.ShapeDtypeStruct((N, NUM_BINS), jnp.float32),
    mesh=plsc.VectorSubcoreMesh(core_axis_name="c", subcore_axis_name="s"),
    compiler_params=pltpu.CompilerParams(needs_layout_passes=False),
    scratch_types=(
        [pltpu.VMEM((RW,), jnp.float32)]
        + [pltpu.VMEM((C, NUM_BINS), jnp.float32) for _ in range(NBUF)]
        + [pltpu.VMEM((2 * C,), jnp.int32) for _ in range(NBUF)]
        + [pltpu.SemaphoreType.DMA for _ in range(NBUF)]
        + [pltpu.SemaphoreType.DMA]
    ),
)
def _two_hot_sc(values_hbm, out_hbm, vals_v, *rest):
    _sc_body(values_hbm, out_hbm, vals_v, *rest)


def kernel(values):
    return _two_hot_sc(values)


# speedup vs baseline: 2.5952x; 1.0077x over previous
import functools

import jax
import jax.numpy as jnp
from jax import lax
from jax.experimental import pallas as pl
from jax.experimental.pallas import tpu as pltpu
from jax.experimental.pallas import tpu_sc as plsc

NUM_BINS = 255
MIN_V = -20.0
MAX_V = 20.0
BIN_WIDTH = (MAX_V - MIN_V) / (NUM_BINS - 1)

N = 262144
NC = 2
NS = 16
NW = NC * NS
RW = N // NW
C = 128
NCH = RW // C
NBUF = 2
L = 16


def _bins(v):
    v = jnp.minimum(jnp.maximum(v, MIN_V), MAX_V)
    norm = (v - MIN_V) / BIN_WIDTH
    lo = jnp.minimum(norm.astype(jnp.int32), NUM_BINS - 1)
    lof = lo.astype(jnp.float32)
    up = jnp.minimum(jnp.where(norm > lof, lo + 1, lo), NUM_BINS - 1)
    return lo, up, norm - lof


def _sc_body(values_hbm, out_hbm, vals_v, *rest):
    bufs = rest[:NBUF]
    sems = rest[NBUF:2 * NBUF]
    vsem = rest[2 * NBUF]

    wid = lax.axis_index("s") * NC + lax.axis_index("c")
    row0 = wid * RW

    vcopy = pltpu.async_copy(values_hbm.at[pl.ds(row0, RW)], vals_v, vsem)

    zeros = jnp.zeros((L,), jnp.float32)
    lane = lax.iota(jnp.int32, L)

    def memset_rows(buf):
        def body(r, carry):
            for g in range(NUM_BINS // L):
                buf[r, pl.ds(g * L, L)] = zeros
            buf[r, pl.ds(NUM_BINS - L, L)] = zeros
            return carry
        lax.fori_loop(0, C, body, 0)

    def chunk_step(chunk, b, clear):
        buf = bufs[b]
        vbase = chunk * C
        ovbase = (chunk - NBUF) * C

        def group(g, carry):
            rows = lane + g * L
            if clear:
                olo, oup, _ = _bins(vals_v[pl.ds(ovbase + g * L, L)])
                plsc.store_scatter(buf, [rows, olo], zeros)
                plsc.store_scatter(buf, [rows, oup], zeros)
            lo, up, uw = _bins(vals_v[pl.ds(vbase + g * L, L)])
            plsc.store_scatter(buf, [rows, lo], 1.0 - uw)
            plsc.addupdate_scatter(buf, [rows, up], uw)
            return carry

        lax.fori_loop(0, C // L, group, 0)

    def issue(chunk, b):
        dst = out_hbm.at[pl.ds(row0 + chunk * C, C)]
        pltpu.async_copy(bufs[b], dst, sems[b])

    def drain(chunk, b):
        dst = out_hbm.at[pl.ds(row0 + chunk * C, C)]
        pltpu.make_async_copy(bufs[b], dst, sems[b]).wait()

    memset_rows(bufs[0])
    vcopy.wait()
    for b in range(NBUF):
        if b:
            memset_rows(bufs[b])
        chunk_step(b, b, clear=False)
        issue(b, b)

    def ring_body(p, carry):
        for b in range(NBUF):
            chunk = p * NBUF + b
            drain(chunk - NBUF, b)
            chunk_step(chunk, b, clear=True)
            issue(chunk, b)
        return carry

    lax.fori_loop(1, NCH // NBUF, ring_body, 0)

    for b in range(NBUF):
        drain(NCH - NBUF + b, b)


@functools.partial(
    pl.kernel,
    out_type=jax.ShapeDtypeStruct((N, NUM_BINS), jnp.float32),
    mesh=plsc.VectorSubcoreMesh(core_axis_name="c", subcore_axis_name="s"),
    compiler_params=pltpu.CompilerParams(needs_layout_passes=False),
    scratch_types=(
        [pltpu.VMEM((RW,), jnp.float32)]
        + [pltpu.VMEM((C, NUM_BINS), jnp.float32) for _ in range(NBUF)]
        + [pltpu.SemaphoreType.DMA for _ in range(NBUF)]
        + [pltpu.SemaphoreType.DMA]
    ),
)
def _two_hot_sc(values_hbm, out_hbm, vals_v, *rest):
    _sc_body(values_hbm, out_hbm, vals_v, *rest)


def kernel(values):
    return _two_hot_sc(values)
